# baseline (speedup 1.0000x reference)
import functools

import jax
import jax.numpy as jnp
from jax import lax
from jax.experimental import pallas as pl
from jax.experimental.pallas import tpu as pltpu
from jax.experimental.pallas import tpu_sc as plsc

N_NODES = 10000
D = 128
E = 320000

NC = 2
NS = 16
NW = NC * NS

CHUNK = 128
CPT = 80
GROUP = 10
EPT = CPT * CHUNK
E_PAD = NW * EPT
ROWS_PER_TILE = 640
ACC_ROWS = NS * ROWS_PER_TILE
TRASH = N_NODES

_mesh = plsc.VectorSubcoreMesh(core_axis_name="c", subcore_axis_name="s")


@functools.partial(
    pl.kernel,
    out_type=jax.ShapeDtypeStruct((NC, ACC_ROWS), jnp.float32),
    mesh=_mesh,
    scratch_types=[
        pltpu.VMEM((CHUNK,), jnp.float32),
        pltpu.VMEM((CPT, CHUNK), jnp.int32),
        pltpu.VMEM_SHARED((ACC_ROWS,), jnp.float32),
    ],
)
def _deg_sc(dst_hbm, zeros_hbm, out_hbm, ones_v, didx, acc):
    cid = lax.axis_index("c")
    sid = lax.axis_index("s")
    wid = sid * NC + cid
    for j in range(CHUNK // 16):
        ones_v[pl.ds(j * 16, 16)] = jnp.ones((16,), jnp.float32)
    pltpu.sync_copy(zeros_hbm, acc.at[pl.ds(sid * ROWS_PER_TILE, ROWS_PER_TILE)])
    pltpu.sync_copy(dst_hbm.at[wid], didx)
    plsc.subcore_barrier()

    def body(c, carry):
        pltpu.sync_copy(ones_v, acc.at[didx.at[c]], add=True)
        return carry

    lax.fori_loop(0, CPT, body, 0)
    plsc.subcore_barrier()
    pltpu.sync_copy(
        acc.at[pl.ds(sid * ROWS_PER_TILE, ROWS_PER_TILE)],
        out_hbm.at[cid, pl.ds(sid * ROWS_PER_TILE, ROWS_PER_TILE)],
    )


@functools.partial(
    pl.kernel,
    out_type=jax.ShapeDtypeStruct((NC, ACC_ROWS, D), jnp.float32),
    mesh=_mesh,
    scratch_types=[
        pltpu.VMEM((GROUP, 2, CHUNK), jnp.int32),
        pltpu.VMEM((CHUNK, D), jnp.float32),
        pltpu.VMEM((CHUNK, D), jnp.float32),
        pltpu.SemaphoreType.DMA,
        pltpu.SemaphoreType.DMA,
        pltpu.VMEM_SHARED((ACC_ROWS, D), jnp.float32),
    ],
)
def _agg_sc(y_hbm, eidx_hbm, zeros_hbm, out_hbm,
            idx_v, rows_a, rows_b, sem_a, sem_b, acc):
    cid = lax.axis_index("c")
    sid = lax.axis_index("s")
    wid = sid * NC + cid
    pltpu.sync_copy(zeros_hbm, acc.at[pl.ds(sid * ROWS_PER_TILE, ROWS_PER_TILE)])
    plsc.subcore_barrier()

    rows = (rows_a, rows_b)
    sems = (sem_a, sem_b)

    def body(g, carry):
        pltpu.sync_copy(eidx_hbm.at[wid, g], idx_v)
        cps = [
            pltpu.async_copy(y_hbm.at[idx_v.at[0, 0]], rows[0], sems[0]),
            pltpu.async_copy(y_hbm.at[idx_v.at[1, 0]], rows[1], sems[1]),
        ]
        for i in range(GROUP):
            cps[i % 2].wait()
            pltpu.sync_copy(rows[i % 2], acc.at[idx_v.at[i, 1]], add=True)
            if i + 2 < GROUP:
                cps[i % 2] = pltpu.async_copy(
                    y_hbm.at[idx_v.at[i + 2, 0]], rows[i % 2], sems[i % 2])
        return carry

    lax.fori_loop(0, CPT // GROUP, body, 0)
    plsc.subcore_barrier()
    pltpu.sync_copy(
        acc.at[pl.ds(sid * ROWS_PER_TILE, ROWS_PER_TILE)],
        out_hbm.at[cid, pl.ds(sid * ROWS_PER_TILE, ROWS_PER_TILE)],
    )


def kernel(x, edge_index, W1, b1, W2, b2):
    src = edge_index[0].astype(jnp.int32)
    dst = edge_index[1].astype(jnp.int32)
    pad = E_PAD - E
    srcp = jnp.concatenate([src, jnp.zeros((pad,), jnp.int32)])
    dstp = jnp.concatenate([dst, jnp.full((pad,), TRASH, jnp.int32)])
    srcp = srcp.reshape(NW, CPT, CHUNK)
    dstp = dstp.reshape(NW, CPT, CHUNK)
    eidx = jnp.stack([srcp, dstp], axis=2)
    eidx_g = eidx.reshape(NW, CPT // GROUP, GROUP, 2, CHUNK)

    zeros1 = jnp.zeros((ROWS_PER_TILE,), jnp.float32)
    zeros2 = jnp.zeros((ROWS_PER_TILE, D), jnp.float32)

    deg_parts = _deg_sc(dstp, zeros1)
    deg = deg_parts[0, :N_NODES] + deg_parts[1, :N_NODES] + 1.0
    dis = lax.rsqrt(deg)[:, None]

    y1 = (x @ W1) * dis
    agg1 = _agg_sc(y1, eidx_g, zeros2)
    h = dis * (agg1[0, :N_NODES] + agg1[1, :N_NODES] + y1) + b1
    h = jnp.maximum(h, 0.0)

    y2 = (h @ W2) * dis
    agg2 = _agg_sc(y2, eidx_g, zeros2)
    return dis * (agg2[0, :N_NODES] + agg2[1, :N_NODES] + y2) + b2
